# baseline (speedup 1.0000x reference)
import jax
import jax.numpy as jnp
from jax import lax
from jax.experimental import pallas as pl
from jax.experimental.pallas import tpu as pltpu
from jax.experimental.pallas import tpu_sc as plsc

B, H, W, P = 4, 512, 512, 50000
MARGIN = 1.0
NPTS = B * P
NC, NS, L = 2, 16, 16
NW = NC * NS
CW = 1568
SW = 1664
PAL = 50048
CHUNK = 128
NFULL = CW // CHUNK
TAILC = CW - NFULL * CHUNK
SUB = CHUNK // L
TSUB = TAILC // L

LN2 = 0.6931471805599453
_LOG2C = (7.283239262169318e-06, 1.4423285361122946, -0.7164483783618765,
          0.45208220030532664, -0.26961100983901826, 0.11592938544152971,
          -0.02429299844067783)


def _softplus(t):
    v = 1.0 + jnp.exp(-t)
    bits = plsc.bitcast(v, jnp.int32)
    e = lax.shift_right_logical(bits, 23) - 127
    m = plsc.bitcast(
        lax.bitwise_or(lax.bitwise_and(bits, 0x7FFFFF), 0x3F800000),
        jnp.float32)
    f = m - 1.0
    p = jnp.float32(_LOG2C[6])
    for co in _LOG2C[5::-1]:
        p = p * f + jnp.float32(co)
    return LN2 * (e.astype(jnp.float32) + p)


def _sc_loss(img, ya, xa, yb, xb, o):
    mesh = plsc.VectorSubcoreMesh(
        core_axis_name="c", subcore_axis_name="s", num_cores=NC, num_subcores=NS
    )

    def body(img_hbm, ya_hbm, xa_hbm, yb_hbm, xb_hbm, o_hbm, out_hbm,
             ya_v, xa_v, yb_v, xb_v, o_v, idxa_v, idxb_v, za_v, zb_v, acc_v,
             drain_v, sem0, sem1, sem2, sem3):
        sems = (sem0, sem1, sem2, sem3)
        wid = lax.axis_index("s") * NC + lax.axis_index("c")
        t = wid * CW
        astart = pl.multiple_of(
            jnp.minimum(lax.shift_left(lax.shift_right_logical(t, 7), 7),
                        PAL - SW), 128)
        sd = pl.multiple_of(jnp.minimum(t, P - CW) - astart, 16)
        dd = t - astart - sd

        cps = [
            pltpu.async_copy(
                src.at[pl.ds(0, B), pl.ds(astart, SW)], dst, sem0)
            for src, dst in ((ya_hbm, ya_v), (xa_hbm, xa_v),
                             (yb_hbm, yb_v), (xb_hbm, xb_v), (o_hbm, o_v))
        ]
        for cp in cps:
            cp.wait()

        def idx_slices(r, boff, off, n16):
            for u in range(n16):
                do = off + u * L
                so = sd + do
                idxa_v[r, pl.ds(do, L)] = (boff + ya_v[r, pl.ds(so, L)] * W
                                           + xa_v[r, pl.ds(so, L)])
                idxb_v[r, pl.ds(do, L)] = (boff + yb_v[r, pl.ds(so, L)] * W
                                           + xb_v[r, pl.ds(so, L)])

        def issue_row(r):
            boff = r * H * W
            sem = sems[r]

            def chunk_body(j, carry):
                co = j * CHUNK
                idx_slices(r, boff, co, SUB)
                pltpu.async_copy(img_hbm.at[idxa_v.at[r, pl.ds(co, CHUNK)]],
                                 za_v.at[r, pl.ds(co, CHUNK)], sem)
                pltpu.async_copy(img_hbm.at[idxb_v.at[r, pl.ds(co, CHUNK)]],
                                 zb_v.at[r, pl.ds(co, CHUNK)], sem)
                return carry

            lax.fori_loop(0, NFULL, chunk_body, 0)
            co = NFULL * CHUNK
            idx_slices(r, boff, co, TSUB)
            pltpu.async_copy(img_hbm.at[idxa_v.at[r, pl.ds(co, TAILC)]],
                             za_v.at[r, pl.ds(co, TAILC)], sem)
            pltpu.async_copy(img_hbm.at[idxb_v.at[r, pl.ds(co, TAILC)]],
                             zb_v.at[r, pl.ds(co, TAILC)], sem)

        def row_wait(r):
            pltpu.make_async_copy(img_hbm.at[pl.ds(0, CW)],
                                  drain_v, sems[r]).wait()
            pltpu.make_async_copy(img_hbm.at[pl.ds(0, CW)],
                                  drain_v, sems[r]).wait()

        def loss_slices(r, acc, off, n16):
            for u in range(n16):
                do = off + u * L
                d = za_v[r, pl.ds(do, L)] - zb_v[r, pl.ds(do, L)]
                o_s = o_v[r, pl.ds(sd + do, L)]
                mask = jnp.abs(o_s)
                tt = jnp.minimum(o_s * d, MARGIN)
                sp = _softplus(tt)
                q_br = jnp.maximum(d * d, MARGIN * MARGIN)
                loss = mask * sp + (1.0 - mask) * q_br
                pos = do + lax.iota(jnp.int32, L)
                zero = loss * 0.0
                acc = acc + jnp.where(pos >= dd, loss, zero)
            return acc

        def loss_row(r, acc):
            def chunk_body(j, acc):
                return loss_slices(r, acc, j * CHUNK, SUB)

            acc = lax.fori_loop(0, NFULL, chunk_body, acc)
            return loss_slices(r, acc, NFULL * CHUNK, TSUB)

        issue_row(0)
        issue_row(1)
        issue_row(2)
        acc = jnp.zeros((L,), jnp.float32)
        row_wait(0)
        acc = loss_row(0, acc)
        row_wait(1)
        acc = loss_row(1, acc)
        issue_row(3)
        row_wait(2)
        acc = loss_row(2, acc)
        row_wait(3)
        acc = loss_row(3, acc)

        acc_v[...] = acc
        pltpu.sync_copy(acc_v, out_hbm.at[pl.ds(wid * L, L)])

    f = pl.kernel(
        body,
        out_type=jax.ShapeDtypeStruct((NW * L,), jnp.float32),
        mesh=mesh,
        compiler_params=pltpu.CompilerParams(needs_layout_passes=False),
        scratch_types=[
            pltpu.VMEM((B, SW), jnp.int32),
            pltpu.VMEM((B, SW), jnp.int32),
            pltpu.VMEM((B, SW), jnp.int32),
            pltpu.VMEM((B, SW), jnp.int32),
            pltpu.VMEM((B, SW), jnp.float32),
            pltpu.VMEM((B, CW), jnp.int32),
            pltpu.VMEM((B, CW), jnp.int32),
            pltpu.VMEM((B, CW), jnp.float32),
            pltpu.VMEM((B, CW), jnp.float32),
            pltpu.VMEM((L,), jnp.float32),
            pltpu.VMEM((CW,), jnp.float32),
            pltpu.SemaphoreType.DMA,
            pltpu.SemaphoreType.DMA,
            pltpu.SemaphoreType.DMA,
            pltpu.SemaphoreType.DMA,
        ],
    )
    return f(img, ya, xa, yb, xb, o)


def _tc_fin_body(p_ref, out_ref):
    out_ref[0, 0] = jnp.sum(p_ref[...]) / NPTS


def _tc_fin(partials):
    return pl.pallas_call(
        _tc_fin_body,
        out_shape=jax.ShapeDtypeStruct((1, 1), jnp.float32),
        out_specs=pl.BlockSpec(memory_space=pltpu.SMEM),
    )(partials)


def kernel(input, x_A, y_A, x_B, y_B, ordinal):
    img = input.reshape(B * H * W)
    partials = _sc_loss(img, y_A.astype(jnp.int32), x_A.astype(jnp.int32),
                        y_B.astype(jnp.int32), x_B.astype(jnp.int32),
                        ordinal.astype(jnp.float32))
    out = _tc_fin(partials.reshape(4, 128))
    return out[0, 0]


# speedup vs baseline: 1.0124x; 1.0124x over previous
import jax
import jax.numpy as jnp
from jax import lax
from jax.experimental import pallas as pl
from jax.experimental.pallas import tpu as pltpu
from jax.experimental.pallas import tpu_sc as plsc

B, H, W, P = 4, 512, 512, 50000
MARGIN = 1.0
NPTS = B * P
NC, NS, L = 2, 16, 16
NW = NC * NS
CW = 1568
SW = 1664
PAL = 50048
CHUNK = 128
NFULL = CW // CHUNK
TAILC = CW - NFULL * CHUNK
SUB = CHUNK // L
TSUB = TAILC // L

LN2 = 0.6931471805599453
_LOG2C = (1.028800281514921e-05 - 127.0, 1.4418068715809396,
          -0.7090304458033646, 0.4165709627027624, -0.19547357654031702,
          0.04612781197702855)


def _softplus(t):
    v = 1.0 + jnp.exp(-t)
    bits = plsc.bitcast(v, jnp.int32)
    e = lax.shift_right_logical(bits, 23)
    m = plsc.bitcast(
        lax.bitwise_or(lax.bitwise_and(bits, 0x7FFFFF), 0x3F800000),
        jnp.float32)
    f = m - 1.0
    p = jnp.float32(_LOG2C[5])
    for co in _LOG2C[4::-1]:
        p = p * f + jnp.float32(co)
    return LN2 * (e.astype(jnp.float32) + p)


def _sc_loss(img, ya, xa, yb, xb, o):
    mesh = plsc.VectorSubcoreMesh(
        core_axis_name="c", subcore_axis_name="s", num_cores=NC, num_subcores=NS
    )

    def body(img_hbm, ya_hbm, xa_hbm, yb_hbm, xb_hbm, o_hbm, out_hbm,
             ya_v, xa_v, yb_v, xb_v, o_v, idxa_v, idxb_v, za_v, zb_v, acc_v,
             drain_v, sem0, sem1, sem2, sem3):
        sems = (sem0, sem1, sem2, sem3)
        wid = lax.axis_index("s") * NC + lax.axis_index("c")
        t = wid * CW
        astart = pl.multiple_of(
            jnp.minimum(lax.shift_left(lax.shift_right_logical(t, 7), 7),
                        PAL - SW), 128)
        sd = pl.multiple_of(jnp.minimum(t, P - CW) - astart, 16)
        dd = t - astart - sd

        cps = [
            pltpu.async_copy(
                src.at[pl.ds(0, B), pl.ds(astart, SW)], dst, sem0)
            for src, dst in ((ya_hbm, ya_v), (xa_hbm, xa_v),
                             (yb_hbm, yb_v), (xb_hbm, xb_v), (o_hbm, o_v))
        ]
        for cp in cps:
            cp.wait()

        def idx_slices(r, boff, off, n16):
            for u in range(n16):
                do = off + u * L
                so = sd + do
                idxa_v[r, pl.ds(do, L)] = (boff + ya_v[r, pl.ds(so, L)] * W
                                           + xa_v[r, pl.ds(so, L)])
                idxb_v[r, pl.ds(do, L)] = (boff + yb_v[r, pl.ds(so, L)] * W
                                           + xb_v[r, pl.ds(so, L)])

        def issue_row(r):
            boff = r * H * W
            sem = sems[r]

            def chunk_body(j, carry):
                co = j * CHUNK
                idx_slices(r, boff, co, SUB)
                pltpu.async_copy(img_hbm.at[idxa_v.at[r, pl.ds(co, CHUNK)]],
                                 za_v.at[r, pl.ds(co, CHUNK)], sem)
                pltpu.async_copy(img_hbm.at[idxb_v.at[r, pl.ds(co, CHUNK)]],
                                 zb_v.at[r, pl.ds(co, CHUNK)], sem)
                return carry

            lax.fori_loop(0, NFULL, chunk_body, 0)
            co = NFULL * CHUNK
            idx_slices(r, boff, co, TSUB)
            pltpu.async_copy(img_hbm.at[idxa_v.at[r, pl.ds(co, TAILC)]],
                             za_v.at[r, pl.ds(co, TAILC)], sem)
            pltpu.async_copy(img_hbm.at[idxb_v.at[r, pl.ds(co, TAILC)]],
                             zb_v.at[r, pl.ds(co, TAILC)], sem)

        def row_wait(r):
            pltpu.make_async_copy(img_hbm.at[pl.ds(0, CW)],
                                  drain_v, sems[r]).wait()
            pltpu.make_async_copy(img_hbm.at[pl.ds(0, CW)],
                                  drain_v, sems[r]).wait()

        def loss_slices(r, acc, off, n16):
            for u in range(n16):
                do = off + u * L
                d = za_v[r, pl.ds(do, L)] - zb_v[r, pl.ds(do, L)]
                o_s = o_v[r, pl.ds(sd + do, L)]
                mask = jnp.abs(o_s)
                tt = jnp.minimum(o_s * d, MARGIN)
                sp = _softplus(tt)
                q_br = jnp.maximum(d * d, MARGIN * MARGIN)
                loss = q_br + mask * (sp - q_br)
                pos = do + lax.iota(jnp.int32, L)
                zero = loss * 0.0
                acc = acc + jnp.where(pos >= dd, loss, zero)
            return acc

        def loss_row(r, acc):
            def chunk_body(j, acc):
                return loss_slices(r, acc, j * CHUNK, SUB)

            acc = lax.fori_loop(0, NFULL, chunk_body, acc)
            return loss_slices(r, acc, NFULL * CHUNK, TSUB)

        issue_row(0)
        issue_row(1)
        acc = jnp.zeros((L,), jnp.float32)
        row_wait(0)
        acc = loss_row(0, acc)
        issue_row(2)
        row_wait(1)
        acc = loss_row(1, acc)
        issue_row(3)
        row_wait(2)
        acc = loss_row(2, acc)
        row_wait(3)
        acc = loss_row(3, acc)

        acc_v[...] = acc
        pltpu.sync_copy(acc_v, out_hbm.at[pl.ds(wid * L, L)])

    f = pl.kernel(
        body,
        out_type=jax.ShapeDtypeStruct((NW * L,), jnp.float32),
        mesh=mesh,
        compiler_params=pltpu.CompilerParams(needs_layout_passes=False),
        scratch_types=[
            pltpu.VMEM((B, SW), jnp.int32),
            pltpu.VMEM((B, SW), jnp.int32),
            pltpu.VMEM((B, SW), jnp.int32),
            pltpu.VMEM((B, SW), jnp.int32),
            pltpu.VMEM((B, SW), jnp.float32),
            pltpu.VMEM((B, CW), jnp.int32),
            pltpu.VMEM((B, CW), jnp.int32),
            pltpu.VMEM((B, CW), jnp.float32),
            pltpu.VMEM((B, CW), jnp.float32),
            pltpu.VMEM((L,), jnp.float32),
            pltpu.VMEM((CW,), jnp.float32),
            pltpu.SemaphoreType.DMA,
            pltpu.SemaphoreType.DMA,
            pltpu.SemaphoreType.DMA,
            pltpu.SemaphoreType.DMA,
        ],
    )
    return f(img, ya, xa, yb, xb, o)


def _tc_fin_body(p_ref, out_ref):
    out_ref[0, 0] = jnp.sum(p_ref[...]) / NPTS


def _tc_fin(partials):
    return pl.pallas_call(
        _tc_fin_body,
        out_shape=jax.ShapeDtypeStruct((1, 1), jnp.float32),
        out_specs=pl.BlockSpec(memory_space=pltpu.SMEM),
    )(partials)


def kernel(input, x_A, y_A, x_B, y_B, ordinal):
    img = input.reshape(B * H * W)
    partials = _sc_loss(img, y_A.astype(jnp.int32), x_A.astype(jnp.int32),
                        y_B.astype(jnp.int32), x_B.astype(jnp.int32),
                        ordinal.astype(jnp.float32))
    out = _tc_fin(partials.reshape(4, 128))
    return out[0, 0]
